# Initial kernel scaffold; baseline (speedup 1.0000x reference)
import jax
import jax.numpy as jnp
from jax.experimental import pallas as pl


def kernel(x, W0, W1, W2, W3, W4, W5, W6, W7, W8):
    raise NotImplementedError("write your pallas kernel here")


# speedup vs baseline: 1.9656x; 1.9656x over previous
import jax
import jax.numpy as jnp
from jax import lax
from jax.experimental import pallas as pl
from jax.experimental.pallas import tpu as pltpu
from jax.experimental.pallas import tpu_sc as plsc

N = 100000
EMB = 512
NF = 9
NC, NS = 2, 16
NW = NC * NS
CHUNK = 64
CHUNKS_PER_W = 50
NPAD = NW * CHUNK * CHUNKS_PER_W
NCHUNKS = NPAD // CHUNK
LANES = 16

G0_DIMS = (119, 2, 2)
G1_DIMS = (4, 11, 12)
G2_DIMS = (9, 5, 8)
G0_ROWS = 119 * 2 * 2
G1_ROWS = 4 * 11 * 12
G2_ROWS = 9 * 5 * 8


def _outer3(a, b, c, dims):
    da, db, dc = dims
    t = jnp.repeat(b, dc, axis=0) + jnp.tile(c, (db, 1))
    return jnp.repeat(a, db * dc, axis=0) + jnp.tile(t, (da, 1))


def _build_body(w0, w1, w2, w3, w4, w5, w6, w7, w8, g0, g1, g2):
    g0[...] = _outer3(w0[...], w7[...], w8[...], G0_DIMS)
    g1[...] = _outer3(w1[...], w2[...], w3[...], G1_DIMS)
    g2[...] = _outer3(w4[...], w5[...], w6[...], G2_DIMS)


_build_tables = pl.pallas_call(
    _build_body,
    out_shape=(
        jax.ShapeDtypeStruct((G0_ROWS, EMB), jnp.float32),
        jax.ShapeDtypeStruct((G1_ROWS, EMB), jnp.float32),
        jax.ShapeDtypeStruct((G2_ROWS, EMB), jnp.float32),
    ),
)


def _sc_body(xr, g0, g1, g2, out, idx_v, code_v, b0, b1, b2, sem):
    wid = lax.axis_index("s") * NC + lax.axis_index("c")

    @pl.loop(0, CHUNKS_PER_W)
    def _chunk(k):
        c = wid * CHUNKS_PER_W + k
        pltpu.sync_copy(xr.at[c], idx_v)
        for j in range(CHUNK // LANES):
            s = pl.ds(j * LANES, LANES)
            code_v[0, s] = (idx_v[0, s] * 4 + idx_v[7, s] * 2) + idx_v[8, s]
            code_v[1, s] = (idx_v[1, s] * 132 + idx_v[2, s] * 12) + idx_v[3, s]
            code_v[2, s] = (idx_v[4, s] * 40 + idx_v[5, s] * 8) + idx_v[6, s]
        d0 = pltpu.async_copy(g0.at[code_v.at[0]], b0, sem)
        d1 = pltpu.async_copy(g1.at[code_v.at[1]], b1, sem)
        d2 = pltpu.async_copy(g2.at[code_v.at[2]], b2, sem)
        d0.wait()
        d1.wait()
        d2.wait()

        @pl.loop(0, CHUNK)
        def _row(r):
            for cc in range(EMB // LANES):
                s = pl.ds(cc * LANES, LANES)
                b0[r, s] = b0[r, s] + b1[r, s] + b2[r, s]

        pltpu.sync_copy(b0, out.at[pl.ds(c * CHUNK, CHUNK)])


_mesh = plsc.VectorSubcoreMesh(core_axis_name="c", subcore_axis_name="s",
                               num_cores=NC, num_subcores=NS)

_sc_call = pl.kernel(
    _sc_body,
    out_type=jax.ShapeDtypeStruct((NPAD, EMB), jnp.float32),
    mesh=_mesh,
    scratch_types=[
        pltpu.VMEM((NF, CHUNK), jnp.int32),
        pltpu.VMEM((3, CHUNK), jnp.int32),
        pltpu.VMEM((CHUNK, EMB), jnp.float32),
        pltpu.VMEM((CHUNK, EMB), jnp.float32),
        pltpu.VMEM((CHUNK, EMB), jnp.float32),
        pltpu.SemaphoreType.DMA,
    ],
)


def kernel(x, W0, W1, W2, W3, W4, W5, W6, W7, W8):
    g0, g1, g2 = _build_tables(W0, W1, W2, W3, W4, W5, W6, W7, W8)
    xpad = jnp.pad(x, ((0, NPAD - N), (0, 0)))
    xr = xpad.T.reshape(NF, NCHUNKS, CHUNK).transpose(1, 0, 2)
    out = _sc_call(xr, g0, g1, g2)
    return out[:N]


# speedup vs baseline: 2.0590x; 1.0475x over previous
import jax
import jax.numpy as jnp
from jax import lax
from jax.experimental import pallas as pl
from jax.experimental.pallas import tpu as pltpu
from jax.experimental.pallas import tpu_sc as plsc

N = 100000
EMB = 512
NF = 9
NC, NS = 2, 16
NW = NC * NS
CHUNK = 32
CHUNKS_PER_W = 98
NPAD = NW * CHUNK * CHUNKS_PER_W
NCHUNKS = NPAD // CHUNK
LANES = 16

G0_DIMS = (119, 2, 2)
G1_DIMS = (4, 11, 12)
G2_DIMS = (9, 5, 8)
G0_ROWS = 119 * 2 * 2
G1_ROWS = 4 * 11 * 12
G2_ROWS = 9 * 5 * 8


def _outer3(a, b, c, dims):
    da, db, dc = dims
    t = jnp.repeat(b, dc, axis=0) + jnp.tile(c, (db, 1))
    return jnp.repeat(a, db * dc, axis=0) + jnp.tile(t, (da, 1))


def _build_body(w0, w1, w2, w3, w4, w5, w6, w7, w8, g0, g1, g2):
    g0[...] = _outer3(w0[...], w7[...], w8[...], G0_DIMS)
    g1[...] = _outer3(w1[...], w2[...], w3[...], G1_DIMS)
    g2[...] = _outer3(w4[...], w5[...], w6[...], G2_DIMS)


_build_tables = pl.pallas_call(
    _build_body,
    out_shape=(
        jax.ShapeDtypeStruct((G0_ROWS, EMB), jnp.float32),
        jax.ShapeDtypeStruct((G1_ROWS, EMB), jnp.float32),
        jax.ShapeDtypeStruct((G2_ROWS, EMB), jnp.float32),
    ),
)


def _sc_body(xr, g0, g1, g2, out,
             idx_a, idx_b, code_a, code_b,
             a0, a1, a2, b0, b1, b2, sem_a, sem_b):
    wid = lax.axis_index("s") * NC + lax.axis_index("c")
    gs = (g0, g1, g2)
    sets = ((idx_a, code_a, (a0, a1, a2), sem_a),
            (idx_b, code_b, (b0, b1, b2), sem_b))

    def prep(k, st):
        idx_v, code_v, bufs, sem = st
        c = wid * CHUNKS_PER_W + k
        pltpu.sync_copy(xr.at[c], idx_v)
        for j in range(CHUNK // LANES):
            s = pl.ds(j * LANES, LANES)
            code_v[0, s] = (idx_v[0, s] * 4 + idx_v[7, s] * 2) + idx_v[8, s]
            code_v[1, s] = (idx_v[1, s] * 132 + idx_v[2, s] * 12) + idx_v[3, s]
            code_v[2, s] = (idx_v[4, s] * 40 + idx_v[5, s] * 8) + idx_v[6, s]
        for g in range(3):
            pltpu.async_copy(gs[g].at[code_v.at[g]], bufs[g], sem)

    def finish(k, st):
        _, _, bufs, sem = st
        c = wid * CHUNKS_PER_W + k
        for g in range(3):
            pltpu.make_async_copy(gs[g].at[pl.ds(0, CHUNK)], bufs[g], sem).wait()
        t0, t1, t2 = bufs

        @pl.loop(0, CHUNK)
        def _row(r):
            for cc in range(EMB // LANES):
                s = pl.ds(cc * LANES, LANES)
                t0[r, s] = t0[r, s] + t1[r, s] + t2[r, s]

        pltpu.sync_copy(t0, out.at[pl.ds(c * CHUNK, CHUNK)])

    prep(0, sets[0])

    @pl.loop(0, CHUNKS_PER_W, step=2)
    def _pipe(k):
        prep(k + 1, sets[1])
        finish(k, sets[0])

        @pl.when(k + 2 < CHUNKS_PER_W)
        def _():
            prep(k + 2, sets[0])

        finish(k + 1, sets[1])


_mesh = plsc.VectorSubcoreMesh(core_axis_name="c", subcore_axis_name="s",
                               num_cores=NC, num_subcores=NS)

_sc_call = pl.kernel(
    _sc_body,
    out_type=jax.ShapeDtypeStruct((NPAD, EMB), jnp.float32),
    mesh=_mesh,
    scratch_types=[
        pltpu.VMEM((NF, CHUNK), jnp.int32),
        pltpu.VMEM((NF, CHUNK), jnp.int32),
        pltpu.VMEM((3, CHUNK), jnp.int32),
        pltpu.VMEM((3, CHUNK), jnp.int32),
        pltpu.VMEM((CHUNK, EMB), jnp.float32),
        pltpu.VMEM((CHUNK, EMB), jnp.float32),
        pltpu.VMEM((CHUNK, EMB), jnp.float32),
        pltpu.VMEM((CHUNK, EMB), jnp.float32),
        pltpu.VMEM((CHUNK, EMB), jnp.float32),
        pltpu.VMEM((CHUNK, EMB), jnp.float32),
        pltpu.SemaphoreType.DMA,
        pltpu.SemaphoreType.DMA,
    ],
)


def kernel(x, W0, W1, W2, W3, W4, W5, W6, W7, W8):
    g0, g1, g2 = _build_tables(W0, W1, W2, W3, W4, W5, W6, W7, W8)
    xpad = jnp.pad(x, ((0, NPAD - N), (0, 0)))
    xr = xpad.T.reshape(NF, NCHUNKS, CHUNK).transpose(1, 0, 2)
    out = _sc_call(xr, g0, g1, g2)
    return out[:N]


# speedup vs baseline: 2.6640x; 1.2938x over previous
import jax
import jax.numpy as jnp
from jax import lax
from jax.experimental import pallas as pl
from jax.experimental.pallas import tpu as pltpu
from jax.experimental.pallas import tpu_sc as plsc

N = 100000
EMB = 512
NF = 9
NC, NS = 2, 16
NW = NC * NS
CHUNK = 48
CHUNKS_PER_W = 66
NPAD = NW * CHUNK * CHUNKS_PER_W
NCHUNKS = NPAD // CHUNK
LANES = 16

A_ROWS = 119 * 8 * 2 * 2
B1_ROWS = 4 * 11 * 12
B2_ROWS = 9 * 5
B_ROWS = B1_ROWS * B2_ROWS
B_BLK = 8


def _outer(parts):
    acc = parts[0]
    for p in parts[1:]:
        acc = jnp.repeat(acc, p.shape[0], axis=0) + jnp.tile(p, (acc.shape[0], 1))
    return acc


def _build1_body(w0, w1, w2, w3, w4, w5, w6, w7, w8, a, b1, b2):
    a[...] = _outer([w0[...], w6[...], w7[...], w8[...]])
    b1[...] = _outer([w1[...], w2[...], w3[...]])
    b2[...] = _outer([w4[...], w5[...]])


_build1 = pl.pallas_call(
    _build1_body,
    out_shape=(
        jax.ShapeDtypeStruct((A_ROWS, EMB), jnp.float32),
        jax.ShapeDtypeStruct((B1_ROWS, EMB), jnp.float32),
        jax.ShapeDtypeStruct((B2_ROWS, EMB), jnp.float32),
    ),
)


def _build2_body(b1_blk, b2, b):
    b[...] = jnp.repeat(b1_blk[...], B2_ROWS, axis=0) + jnp.tile(b2[...], (B_BLK, 1))


_build2 = pl.pallas_call(
    _build2_body,
    grid=(B1_ROWS // B_BLK,),
    in_specs=[
        pl.BlockSpec((B_BLK, EMB), lambda i: (i, 0)),
        pl.BlockSpec((B2_ROWS, EMB), lambda i: (0, 0)),
    ],
    out_specs=pl.BlockSpec((B_BLK * B2_ROWS, EMB), lambda i: (i, 0)),
    out_shape=jax.ShapeDtypeStruct((B_ROWS, EMB), jnp.float32),
)


def _sc_body(xr, ga, gb, out,
             idx_a, idx_b, code_a, code_b,
             a0, a1, b0, b1, sem_a, sem_b):
    wid = lax.axis_index("s") * NC + lax.axis_index("c")
    sets = ((idx_a, code_a, (a0, a1), sem_a),
            (idx_b, code_b, (b0, b1), sem_b))

    def prep(k, st):
        idx_v, code_v, bufs, sem = st
        c = wid * CHUNKS_PER_W + k
        pltpu.sync_copy(xr.at[c], idx_v)
        for j in range(CHUNK // LANES):
            s = pl.ds(j * LANES, LANES)
            code_v[0, s] = ((idx_v[0, s] * 32 + idx_v[6, s] * 4)
                            + (idx_v[7, s] * 2 + idx_v[8, s]))
            code_v[1, s] = ((idx_v[1, s] * 5940 + idx_v[2, s] * 540)
                            + (idx_v[3, s] * 45 + idx_v[4, s] * 5 + idx_v[5, s]))
        pltpu.async_copy(ga.at[code_v.at[0]], bufs[0], sem)
        pltpu.async_copy(gb.at[code_v.at[1]], bufs[1], sem)

    def finish(k, st):
        _, _, bufs, sem = st
        c = wid * CHUNKS_PER_W + k
        pltpu.make_async_copy(ga.at[pl.ds(0, CHUNK)], bufs[0], sem).wait()
        pltpu.make_async_copy(gb.at[pl.ds(0, CHUNK)], bufs[1], sem).wait()
        t0, t1 = bufs

        @pl.loop(0, CHUNK)
        def _row(r):
            for cc in range(EMB // LANES):
                s = pl.ds(cc * LANES, LANES)
                t0[r, s] = t0[r, s] + t1[r, s]

        pltpu.sync_copy(t0, out.at[pl.ds(c * CHUNK, CHUNK)])

    prep(0, sets[0])

    @pl.loop(0, CHUNKS_PER_W, step=2)
    def _pipe(k):
        prep(k + 1, sets[1])
        finish(k, sets[0])

        @pl.when(k + 2 < CHUNKS_PER_W)
        def _():
            prep(k + 2, sets[0])

        finish(k + 1, sets[1])


_mesh = plsc.VectorSubcoreMesh(core_axis_name="c", subcore_axis_name="s",
                               num_cores=NC, num_subcores=NS)

_sc_call = pl.kernel(
    _sc_body,
    out_type=jax.ShapeDtypeStruct((NPAD, EMB), jnp.float32),
    mesh=_mesh,
    scratch_types=[
        pltpu.VMEM((NF, CHUNK), jnp.int32),
        pltpu.VMEM((NF, CHUNK), jnp.int32),
        pltpu.VMEM((2, CHUNK), jnp.int32),
        pltpu.VMEM((2, CHUNK), jnp.int32),
        pltpu.VMEM((CHUNK, EMB), jnp.float32),
        pltpu.VMEM((CHUNK, EMB), jnp.float32),
        pltpu.VMEM((CHUNK, EMB), jnp.float32),
        pltpu.VMEM((CHUNK, EMB), jnp.float32),
        pltpu.SemaphoreType.DMA,
        pltpu.SemaphoreType.DMA,
    ],
)


def kernel(x, W0, W1, W2, W3, W4, W5, W6, W7, W8):
    a, b1, b2 = _build1(W0, W1, W2, W3, W4, W5, W6, W7, W8)
    b = _build2(b1, b2)
    xpad = jnp.pad(x, ((0, NPAD - N), (0, 0)))
    xr = xpad.T.reshape(NF, NCHUNKS, CHUNK).transpose(1, 0, 2)
    out = _sc_call(xr, a, b)
    return out[:N]


# speedup vs baseline: 4.1453x; 1.5560x over previous
import jax
import jax.numpy as jnp
from jax import lax
from jax.experimental import pallas as pl
from jax.experimental.pallas import tpu as pltpu
from jax.experimental.pallas import tpu_sc as plsc

N = 100000
EMB = 512
NF = 9
NC, NS = 2, 16
NW = NC * NS
CHUNK = 48
CHUNKS_PER_W = 34
NPAD = NW * CHUNK * CHUNKS_PER_W
NCHUNKS = NPAD // CHUNK
LANES = 16

TC_BN = 512
NTC = 96 * TC_BN
FEAT_DIMS = (119, 4, 11, 12, 9, 5, 8, 2, 2)
KTOT = sum(FEAT_DIMS)
KP = 256
FEAT_OFF = tuple(sum(FEAT_DIMS[:i]) for i in range(NF))

A_ROWS = 119 * 8 * 2 * 2
B1_ROWS = 4 * 11 * 12
B2_ROWS = 9 * 5
B_ROWS = B1_ROWS * B2_ROWS
B_BLK = 8


def _outer(parts):
    acc = parts[0]
    for p in parts[1:]:
        acc = jnp.repeat(acc, p.shape[0], axis=0) + jnp.tile(p, (acc.shape[0], 1))
    return acc


def _build1_body(w0, w1, w2, w3, w4, w5, w6, w7, w8, a, b1, b2):
    a[...] = _outer([w0[...], w6[...], w7[...], w8[...]])
    b1[...] = _outer([w1[...], w2[...], w3[...]])
    b2[...] = _outer([w4[...], w5[...]])


_build1 = pl.pallas_call(
    _build1_body,
    out_shape=(
        jax.ShapeDtypeStruct((A_ROWS, EMB), jnp.float32),
        jax.ShapeDtypeStruct((B1_ROWS, EMB), jnp.float32),
        jax.ShapeDtypeStruct((B2_ROWS, EMB), jnp.float32),
    ),
)


def _build2_body(b1_blk, b2, b):
    b[...] = jnp.repeat(b1_blk[...], B2_ROWS, axis=0) + jnp.tile(b2[...], (B_BLK, 1))


_build2 = pl.pallas_call(
    _build2_body,
    grid=(B1_ROWS // B_BLK,),
    in_specs=[
        pl.BlockSpec((B_BLK, EMB), lambda i: (i, 0)),
        pl.BlockSpec((B2_ROWS, EMB), lambda i: (0, 0)),
    ],
    out_specs=pl.BlockSpec((B_BLK * B2_ROWS, EMB), lambda i: (i, 0)),
    out_shape=jax.ShapeDtypeStruct((B_ROWS, EMB), jnp.float32),
)


def _tc_body(xt_ref, w_ref, o_ref):
    xb = xt_ref[...]
    iota0 = lax.broadcasted_iota(jnp.int32, (KP, TC_BN), 0)
    oh = (iota0 == xb[0:1, :] + FEAT_OFF[0]).astype(jnp.float32)
    for f in range(1, NF):
        oh += (iota0 == xb[f:f + 1, :] + FEAT_OFF[f]).astype(jnp.float32)
    o_ref[...] = lax.dot_general(oh, w_ref[...], (((0,), (0,)), ((), ())),
                                 preferred_element_type=jnp.float32)


_tc_call = pl.pallas_call(
    _tc_body,
    grid=(NTC // TC_BN,),
    in_specs=[
        pl.BlockSpec((NF, TC_BN), lambda i: (0, i)),
        pl.BlockSpec((KP, EMB), lambda i: (0, 0)),
    ],
    out_specs=pl.BlockSpec((TC_BN, EMB), lambda i: (i, 0)),
    out_shape=jax.ShapeDtypeStruct((NTC, EMB), jnp.float32),
)


def _sc_body(xr, ga, gb, out,
             idx_a, idx_b, code_a, code_b,
             a0, a1, b0, b1, sem_a, sem_b):
    wid = lax.axis_index("s") * NC + lax.axis_index("c")
    sets = ((idx_a, code_a, (a0, a1), sem_a),
            (idx_b, code_b, (b0, b1), sem_b))

    def prep(k, st):
        idx_v, code_v, bufs, sem = st
        c = wid * CHUNKS_PER_W + k
        pltpu.sync_copy(xr.at[c], idx_v)
        for j in range(CHUNK // LANES):
            s = pl.ds(j * LANES, LANES)
            code_v[0, s] = ((idx_v[0, s] * 32 + idx_v[6, s] * 4)
                            + (idx_v[7, s] * 2 + idx_v[8, s]))
            code_v[1, s] = ((idx_v[1, s] * 5940 + idx_v[2, s] * 540)
                            + (idx_v[3, s] * 45 + idx_v[4, s] * 5 + idx_v[5, s]))
        pltpu.async_copy(ga.at[code_v.at[0]], bufs[0], sem)
        pltpu.async_copy(gb.at[code_v.at[1]], bufs[1], sem)

    def finish(k, st):
        _, _, bufs, sem = st
        c = wid * CHUNKS_PER_W + k
        pltpu.make_async_copy(ga.at[pl.ds(0, CHUNK)], bufs[0], sem).wait()
        pltpu.make_async_copy(gb.at[pl.ds(0, CHUNK)], bufs[1], sem).wait()
        t0, t1 = bufs

        @pl.loop(0, CHUNK)
        def _row(r):
            for cc in range(EMB // LANES):
                s = pl.ds(cc * LANES, LANES)
                t0[r, s] = t0[r, s] + t1[r, s]

        pltpu.sync_copy(t0, out.at[pl.ds(c * CHUNK, CHUNK)])

    prep(0, sets[0])

    @pl.loop(0, CHUNKS_PER_W, step=2)
    def _pipe(k):
        prep(k + 1, sets[1])
        finish(k, sets[0])

        @pl.when(k + 2 < CHUNKS_PER_W)
        def _():
            prep(k + 2, sets[0])

        finish(k + 1, sets[1])


_mesh = plsc.VectorSubcoreMesh(core_axis_name="c", subcore_axis_name="s",
                               num_cores=NC, num_subcores=NS)

_sc_call = pl.kernel(
    _sc_body,
    out_type=jax.ShapeDtypeStruct((NPAD, EMB), jnp.float32),
    mesh=_mesh,
    scratch_types=[
        pltpu.VMEM((NF, CHUNK), jnp.int32),
        pltpu.VMEM((NF, CHUNK), jnp.int32),
        pltpu.VMEM((2, CHUNK), jnp.int32),
        pltpu.VMEM((2, CHUNK), jnp.int32),
        pltpu.VMEM((CHUNK, EMB), jnp.float32),
        pltpu.VMEM((CHUNK, EMB), jnp.float32),
        pltpu.VMEM((CHUNK, EMB), jnp.float32),
        pltpu.VMEM((CHUNK, EMB), jnp.float32),
        pltpu.SemaphoreType.DMA,
        pltpu.SemaphoreType.DMA,
    ],
)


def kernel(x, W0, W1, W2, W3, W4, W5, W6, W7, W8):
    a, b1, b2 = _build1(W0, W1, W2, W3, W4, W5, W6, W7, W8)
    b = _build2(b1, b2)
    xsc = jnp.pad(x[NTC:], ((0, NPAD - (N - NTC)), (0, 0)))
    xr = xsc.T.reshape(NF, NCHUNKS, CHUNK).transpose(1, 0, 2)
    out_sc = _sc_call(xr, a, b)
    wcat = jnp.concatenate([W0, W1, W2, W3, W4, W5, W6, W7, W8], axis=0)
    wcat = jnp.pad(wcat, ((0, KP - KTOT), (0, 0)))
    out_tc = _tc_call(x[:NTC].T, wcat)
    return jnp.concatenate([out_tc, out_sc[:N - NTC]], axis=0)
